# baseline (speedup 1.0000x reference)
import jax
import jax.numpy as jnp
from jax import lax
from jax.experimental import pallas as pl
from jax.experimental.pallas import tpu as pltpu
from jax.experimental.pallas import tpu_sc as plsc

B, T = 4096, 50
D = 128
TP = 56
NC, NS = 2, 16
NW = NC * NS
S = 4
BS = B // S
BW = BS // NW
NBUF = 8


def _emb_body(idx_hbm, table_hbm, out_hbm, idx_v, bufs, sem_g, sem_s):
    wid = lax.axis_index("s") * NC + lax.axis_index("c")
    b0 = wid * BW

    pltpu.sync_copy(idx_hbm.at[pl.ds(b0 * TP, BW * TP)], idx_v)

    def gather(c, j):
        pltpu.async_copy(
            table_hbm.at[idx_v.at[pl.ds(c * TP, T)]], bufs.at[j], sem_g.at[j])

    def scatter(c, j):
        pltpu.async_copy(bufs.at[j], out_hbm.at[b0 + c], sem_s.at[j])

    def wait_g(j):
        pltpu.make_async_copy(out_hbm.at[0], bufs.at[j], sem_g.at[j]).wait()

    def wait_s(j):
        pltpu.make_async_copy(bufs.at[j], out_hbm.at[0], sem_s.at[j]).wait()

    for j in range(NBUF):
        gather(j, j)

    def body(g, carry):
        c = g * NBUF
        for j in range(NBUF):
            wait_g(j)
            scatter(c + j, j)
        for j in range(NBUF):
            wait_s(j)
            gather(c + NBUF + j, j)
        return carry

    lax.fori_loop(0, BW // NBUF - 1, body, 0)

    c = BW - NBUF
    for j in range(NBUF):
        wait_g(j)
        scatter(c + j, j)
    for j in range(NBUF):
        wait_s(j)


def _slice_lookup(idx_pad_slice, weights):
    mesh = plsc.VectorSubcoreMesh(core_axis_name="c", subcore_axis_name="s")
    k = pl.kernel(
        _emb_body,
        mesh=mesh,
        out_type=jax.ShapeDtypeStruct((BS, T, D), jnp.float32),
        scratch_types=[
            pltpu.VMEM((BW * TP,), jnp.int32),
            pltpu.VMEM((NBUF, T, D), jnp.float32),
            pltpu.SemaphoreType.DMA((NBUF,)),
            pltpu.SemaphoreType.DMA((NBUF,)),
        ],
        compiler_params=pltpu.CompilerParams(use_tc_tiling_on_sc=True),
    )
    return k(idx_pad_slice, weights)


def kernel(token_ids, weights):
    ids = token_ids.astype(jnp.int32)
    idx_pad = jnp.pad(ids, ((0, 0), (0, TP - T))).reshape(-1)
    outs = [
        _slice_lookup(
            lax.dynamic_slice(idx_pad, (s * BS * TP,), (BS * TP,)), weights)
        for s in range(S)
    ]
    return jnp.concatenate(outs, axis=0)


# speedup vs baseline: 3.1535x; 3.1535x over previous
import jax
import jax.numpy as jnp
from jax import lax
from jax.experimental import pallas as pl
from jax.experimental.pallas import tpu as pltpu
from jax.experimental.pallas import tpu_sc as plsc

B, T = 4096, 50
D = 128
TPAD = 56
NC, NS = 2, 16
NW = NC * NS
BW = B // NW
NBUF = 5
NGROUP = T // NBUF


def _emb_body(idx_hbm, table_hbm, out_hbm, idx_v, bufs, sem_g, sem_s):
    wid = lax.axis_index("s") * NC + lax.axis_index("c")
    b0 = wid * BW

    pltpu.sync_copy(idx_hbm.at[:, pl.ds(b0, BW)], idx_v)

    def gather(t, j):
        pltpu.async_copy(table_hbm.at[idx_v.at[t]], bufs.at[j], sem_g.at[j])

    def scatter(t, j):
        pltpu.async_copy(bufs.at[j], out_hbm.at[t, pl.ds(b0, BW)], sem_s.at[j])

    def wait_g(j):
        pltpu.make_async_copy(out_hbm.at[0, pl.ds(0, BW)], bufs.at[j],
                              sem_g.at[j]).wait()

    def wait_s(j):
        pltpu.make_async_copy(bufs.at[j], out_hbm.at[0, pl.ds(0, BW)],
                              sem_s.at[j]).wait()

    for j in range(NBUF):
        gather(j, j)

    def body(g, carry):
        t = g * NBUF
        for j in range(NBUF):
            wait_g(j)
            scatter(t + j, j)
        for j in range(NBUF):
            wait_s(j)
            gather(t + NBUF + j, j)
        return carry

    lax.fori_loop(0, NGROUP - 1, body, 0)

    t = T - NBUF
    for j in range(NBUF):
        wait_g(j)
        scatter(t + j, j)
    for j in range(NBUF):
        wait_s(j)


def _embedding_lookup(idx_t, weights):
    mesh = plsc.VectorSubcoreMesh(core_axis_name="c", subcore_axis_name="s")
    k = pl.kernel(
        _emb_body,
        mesh=mesh,
        out_type=jax.ShapeDtypeStruct((T, B, D), jnp.float32),
        scratch_types=[
            pltpu.VMEM((TPAD, BW), jnp.int32),
            pltpu.VMEM((NBUF, BW, D), jnp.float32),
            pltpu.SemaphoreType.DMA((NBUF,)),
            pltpu.SemaphoreType.DMA((NBUF,)),
        ],
        compiler_params=pltpu.CompilerParams(use_tc_tiling_on_sc=True),
    )
    return k(idx_t, weights)


def kernel(token_ids, weights):
    ids_t = jnp.pad(token_ids.astype(jnp.int32).T, ((0, TPAD - T), (0, 0)))
    out_t = _embedding_lookup(ids_t, weights)
    return jnp.transpose(out_t, (1, 0, 2))


# speedup vs baseline: 3.1788x; 1.0080x over previous
import jax
import jax.numpy as jnp
from jax import lax
from jax.experimental import pallas as pl
from jax.experimental.pallas import tpu as pltpu
from jax.experimental.pallas import tpu_sc as plsc

B, T = 4096, 50
D = 128
TPAD = 56
NC, NS = 2, 16
NW = NC * NS
BW = B // NW
NBUF = 5
NGROUP = T // NBUF


def _emb_body(idx_hbm, table_hbm, out_hbm, idx_v, bufs, sem_g, sem_s):
    wid = lax.axis_index("s") * NC + lax.axis_index("c")
    b0 = wid * BW

    pltpu.sync_copy(idx_hbm.at[:, pl.ds(b0, BW)], idx_v)

    def gather(t, j):
        pltpu.async_copy(table_hbm.at[idx_v.at[t]], bufs.at[j], sem_g.at[j])

    def scatter(t, j):
        pltpu.async_copy(bufs.at[j], out_hbm.at[t, pl.ds(b0, BW)], sem_s.at[j])

    def wait_g(j):
        pltpu.make_async_copy(out_hbm.at[0, pl.ds(0, BW)], bufs.at[j],
                              sem_g.at[j]).wait()

    def wait_s(j):
        pltpu.make_async_copy(bufs.at[j], out_hbm.at[0, pl.ds(0, BW)],
                              sem_s.at[j]).wait()

    for j in range(NBUF):
        gather(j, j)

    def body(g, carry):
        t = g * NBUF
        for j in range(NBUF):
            wait_g(j)
            scatter(t + j, j)
        for j in range(NBUF):
            wait_s(j)
            gather(t + NBUF + j, j)
        return carry

    lax.fori_loop(0, NGROUP - 1, body, 0)

    t = T - NBUF
    for j in range(NBUF):
        wait_g(j)
        scatter(t + j, j)
    for j in range(NBUF):
        wait_s(j)


def _embedding_lookup(idx_t, weights):
    mesh = plsc.VectorSubcoreMesh(core_axis_name="c", subcore_axis_name="s")
    k = pl.kernel(
        _emb_body,
        mesh=mesh,
        out_type=jax.ShapeDtypeStruct((T, B, D), jnp.float32),
        scratch_types=[
            pltpu.VMEM((T, BW), jnp.int32),
            pltpu.VMEM((NBUF, BW, D), jnp.float32),
            pltpu.SemaphoreType.DMA((NBUF,)),
            pltpu.SemaphoreType.DMA((NBUF,)),
        ],
        compiler_params=pltpu.CompilerParams(use_tc_tiling_on_sc=True),
    )
    return k(idx_t, weights)


def kernel(token_ids, weights):
    ids_t = token_ids.astype(jnp.int32).T
    out_t = _embedding_lookup(ids_t, weights)
    return jnp.transpose(out_t, (1, 0, 2))
